# Initial kernel scaffold; baseline (speedup 1.0000x reference)
import jax
import jax.numpy as jnp
from jax.experimental import pallas as pl


def kernel(var_feats, constr_feats, edge_index, edge_attr, W1, b1, g1, bt1, W2, b2, W3, b3, g2, bt2, W4, b4):
    raise NotImplementedError("write your pallas kernel here")


# speedup vs baseline: 2.0973x; 2.0973x over previous
import functools

import jax
import jax.numpy as jnp
from jax import lax
from jax.experimental import pallas as pl
from jax.experimental.pallas import tpu as pltpu
from jax.experimental.pallas import tpu_sc as plsc

N_NODES = 10000
N_EDGES = 320000
D = 128
DH = D // 2
EPS = 1e-5

NC = 2
NS = 16
EB = 128
NBLK = 160
NE_PAD = NS * NBLK * EB
ZROWS = 80
NCHUNK = N_NODES // ZROWS


def _sc_scatter_build():
  mesh = plsc.VectorSubcoreMesh(core_axis_name="c", subcore_axis_name="s")
  half = jax.ShapeDtypeStruct((N_NODES, DH), jnp.float32)
  out_t = (half, half, half, half)

  @functools.partial(
      pl.kernel,
      out_type=out_t,
      mesh=mesh,
      compiler_params=pltpu.CompilerParams(use_tc_tiling_on_sc=False),
      scratch_types=[
          pltpu.VMEM((NBLK, EB), jnp.int32),
          pltpu.VMEM((NBLK, EB), jnp.int32),
          pltpu.VMEM((NBLK * EB,), jnp.float32),
          pltpu.VMEM((EB, DH), jnp.float32),
          pltpu.VMEM((ZROWS, DH), jnp.float32),
          pltpu.VMEM_SHARED((N_NODES, DH), jnp.float32),
      ],
  )
  def sc_kernel(var_lo, var_hi, constr_lo, constr_hi,
                src_hbm, dst_hbm, attr_hbm,
                vagg_lo, vagg_hi, cagg_lo, cagg_hi,
                gidx_v, sidx_v, attr_v, rows_v, zbuf_v, acc_sh):
    c = lax.axis_index("c")
    s = lax.axis_index("s")

    def zfill(i, _):
      for g in range(DH // 16):
        zbuf_v[i, pl.ds(g * 16, 16)] = jnp.zeros((16,), jnp.float32)
      return 0
    lax.fori_loop(0, ZROWS, zfill, 0)

    @pl.when(c == 0)
    def _():
      pltpu.sync_copy(src_hbm.at[s], gidx_v)
      pltpu.sync_copy(dst_hbm.at[s], sidx_v)

    @pl.when(c == 1)
    def _():
      pltpu.sync_copy(dst_hbm.at[s], gidx_v)
      pltpu.sync_copy(src_hbm.at[s], sidx_v)

    pltpu.sync_copy(attr_hbm.at[s], attr_v)

    def zero_acc():
      def zchunk(k, _):
        idx = s + k * NS
        @pl.when(idx < NCHUNK)
        def _():
          pltpu.sync_copy(zbuf_v, acc_sh.at[pl.ds(idx * ZROWS, ZROWS)])
        return 0
      lax.fori_loop(0, (NCHUNK + NS - 1) // NS, zchunk, 0)

    def process(table_hbm, out_hbm):
      def block(j, _):
        pltpu.sync_copy(table_hbm.at[gidx_v.at[j]], rows_v)

        def mul(gr, _):
          base = gr * 16
          a16 = attr_v[pl.ds(j * EB + base, 16)]
          for e16 in range(16):
            a = a16.at[jnp.full((16,), e16, jnp.int32)].get(
                mode="promise_in_bounds")
            for g in range(DH // 16):
              sl = pl.ds(g * 16, 16)
              rows_v[base + e16, sl] = rows_v[base + e16, sl] * a
          return 0
        lax.fori_loop(0, EB // 16, mul, 0)

        pltpu.sync_copy(rows_v, acc_sh.at[sidx_v.at[j]], add=True)
        return 0
      lax.fori_loop(0, NBLK, block, 0)
      plsc.subcore_barrier()

      def wb(k, _):
        idx = s + k * NS
        @pl.when(idx < NCHUNK)
        def _():
          off = idx * ZROWS
          pltpu.sync_copy(acc_sh.at[pl.ds(off, ZROWS)],
                          out_hbm.at[pl.ds(off, ZROWS)])
        return 0
      lax.fori_loop(0, (NCHUNK + NS - 1) // NS, wb, 0)

    tables = ((var_lo, var_hi), (constr_lo, constr_hi))
    outs = ((cagg_lo, cagg_hi), (vagg_lo, vagg_hi))
    for h in range(2):
      zero_acc()
      plsc.subcore_barrier()

      @pl.when(c == 0)
      def _():
        process(tables[0][h], outs[0][h])

      @pl.when(c == 1)
      def _():
        process(tables[1][h], outs[1][h])

  return sc_kernel


_sc_scatter = _sc_scatter_build()


def _mlp_body(vf, cf, va, ca,
              w1a, w1b, b1, g1, bt1, w2, b2,
              w3a, w3b, b3, g2, bt2, w4, b4,
              ov, oc):
  def upd(x_ref, agg_ref, wa, wb, b, g, bt, wo, bo, out_ref):
    h = jnp.dot(x_ref[...], wa[...], preferred_element_type=jnp.float32)
    h = h + jnp.dot(agg_ref[...], wb[...], preferred_element_type=jnp.float32)
    h = h + b[...]
    mu = jnp.mean(h, axis=0, keepdims=True)
    var = jnp.mean(h * h, axis=0, keepdims=True) - mu * mu
    hn = (h - mu) * lax.rsqrt(var + EPS) * g[...] + bt[...]
    hr = jnp.maximum(hn, 0.0)
    out_ref[...] = (x_ref[...] +
                    jnp.dot(hr, wo[...], preferred_element_type=jnp.float32) +
                    bo[...])

  upd(vf, va, w1a, w1b, b1, g1, bt1, w2, b2, ov)
  upd(cf, ca, w3a, w3b, b3, g2, bt2, w4, b4, oc)


def kernel(var_feats, constr_feats, edge_index, edge_attr,
           W1, b1, g1, bt1, W2, b2, W3, b3, g2, bt2, W4, b4):
  src = edge_index[0]
  dst = edge_index[1]
  pad = NE_PAD - N_EDGES
  src_r = jnp.concatenate([src, jnp.zeros((pad,), jnp.int32)]
                          ).reshape(NS, NBLK, EB)
  dst_r = jnp.concatenate([dst, jnp.zeros((pad,), jnp.int32)]
                          ).reshape(NS, NBLK, EB)
  attr_r = jnp.concatenate([edge_attr, jnp.zeros((pad,), jnp.float32)]
                           ).reshape(NS, NBLK * EB)

  vagg_lo, vagg_hi, cagg_lo, cagg_hi = _sc_scatter(
      var_feats[:, :DH], var_feats[:, DH:],
      constr_feats[:, :DH], constr_feats[:, DH:],
      src_r, dst_r, attr_r)
  var_agg = jnp.concatenate([vagg_lo, vagg_hi], axis=1)
  constr_agg = jnp.concatenate([cagg_lo, cagg_hi], axis=1)

  row = lambda v: v.reshape(1, D)
  out_shape = (jax.ShapeDtypeStruct((N_NODES, D), jnp.float32),
               jax.ShapeDtypeStruct((N_NODES, D), jnp.float32))
  var_updated, constr_updated = pl.pallas_call(
      _mlp_body, out_shape=out_shape)(
          var_feats, constr_feats, var_agg, constr_agg,
          W1[:, :D].T, W1[:, D:].T, row(b1), row(g1), row(bt1), W2.T, row(b2),
          W3[:, :D].T, W3[:, D:].T, row(b3), row(g2), row(bt2), W4.T, row(b4))
  return (var_updated, constr_updated)


# speedup vs baseline: 3.0681x; 1.4629x over previous
import functools

import jax
import jax.numpy as jnp
from jax import lax
from jax.experimental import pallas as pl
from jax.experimental.pallas import tpu as pltpu
from jax.experimental.pallas import tpu_sc as plsc

N_NODES = 10000
N_EDGES = 320000
D = 128
DH = D // 2
EPS = 1e-5

NC = 2
NS = 16
EB = 128
NBLK = 160
NE_PAD = NS * NBLK * EB
ZROWS = 80
NCHUNK = N_NODES // ZROWS


def _sc_scatter_build():
  mesh = plsc.VectorSubcoreMesh(core_axis_name="c", subcore_axis_name="s")
  half = jax.ShapeDtypeStruct((N_NODES, DH), jnp.float32)
  out_t = (half, half, half, half)

  @functools.partial(
      pl.kernel,
      out_type=out_t,
      mesh=mesh,
      compiler_params=pltpu.CompilerParams(use_tc_tiling_on_sc=False),
      scratch_types=[
          pltpu.VMEM((NBLK, EB), jnp.int32),
          pltpu.VMEM((NBLK, EB), jnp.int32),
          pltpu.VMEM((NBLK * EB,), jnp.float32),
          pltpu.VMEM((2, EB, DH), jnp.float32),
          pltpu.VMEM((ZROWS, DH), jnp.float32),
          pltpu.VMEM_SHARED((N_NODES, DH), jnp.float32),
          pltpu.SemaphoreType.DMA((2,)),
          pltpu.SemaphoreType.DMA((2,)),
      ],
  )
  def sc_kernel(var_lo, var_hi, constr_lo, constr_hi,
                src_hbm, dst_hbm, attr_hbm,
                vagg_lo, vagg_hi, cagg_lo, cagg_hi,
                gidx_v, sidx_v, attr_v, rows_v, zbuf_v, acc_sh,
                sem_g, sem_s):
    c = lax.axis_index("c")
    s = lax.axis_index("s")

    def zfill(i, _):
      for g in range(DH // 16):
        zbuf_v[i, pl.ds(g * 16, 16)] = jnp.zeros((16,), jnp.float32)
      return 0
    lax.fori_loop(0, ZROWS, zfill, 0)

    @pl.when(c == 0)
    def _():
      pltpu.sync_copy(src_hbm.at[s], gidx_v)
      pltpu.sync_copy(dst_hbm.at[s], sidx_v)

    @pl.when(c == 1)
    def _():
      pltpu.sync_copy(dst_hbm.at[s], gidx_v)
      pltpu.sync_copy(src_hbm.at[s], sidx_v)

    pltpu.sync_copy(attr_hbm.at[s], attr_v)

    def zero_acc():
      def zchunk(k, _):
        idx = s + k * NS
        @pl.when(idx < NCHUNK)
        def _():
          pltpu.sync_copy(zbuf_v, acc_sh.at[pl.ds(idx * ZROWS, ZROWS)])
        return 0
      lax.fori_loop(0, (NCHUNK + NS - 1) // NS, zchunk, 0)

    def process(table_hbm, out_hbm):
      def g_start(j, b):
        pltpu.async_copy(table_hbm.at[gidx_v.at[j]], rows_v.at[b],
                         sem_g.at[b])

      def g_wait(j, b):
        pltpu.make_async_copy(table_hbm.at[gidx_v.at[j]], rows_v.at[b],
                              sem_g.at[b]).wait()

      def s_start(j, b):
        pltpu.async_copy(rows_v.at[b], acc_sh.at[sidx_v.at[j]],
                         sem_s.at[b], add=True)

      def s_wait(j, b):
        pltpu.make_async_copy(rows_v.at[b], acc_sh.at[sidx_v.at[j]],
                              sem_s.at[b]).wait()

      g_start(0, 0)

      def block(j, _):
        b = jnp.bitwise_and(j, 1)
        nb = jnp.bitwise_and(j + 1, 1)

        @pl.when(j >= 1)
        def _():
          s_wait(j - 1, nb)

        @pl.when(j + 1 < NBLK)
        def _():
          g_start(j + 1, nb)

        g_wait(j, b)

        def mul(gr, _):
          base = gr * 16
          a16 = attr_v[pl.ds(j * EB + base, 16)]
          for e16 in range(16):
            a = a16.at[jnp.full((16,), e16, jnp.int32)].get(
                mode="promise_in_bounds")
            for g in range(DH // 16):
              sl = pl.ds(g * 16, 16)
              rows_v[b, base + e16, sl] = rows_v[b, base + e16, sl] * a
          return 0
        lax.fori_loop(0, EB // 16, mul, 0)

        s_start(j, b)
        return 0
      lax.fori_loop(0, NBLK, block, 0)
      s_wait(NBLK - 1, (NBLK - 1) % 2)
      plsc.subcore_barrier()

      def wb(k, _):
        idx = s + k * NS
        @pl.when(idx < NCHUNK)
        def _():
          off = idx * ZROWS
          pltpu.sync_copy(acc_sh.at[pl.ds(off, ZROWS)],
                          out_hbm.at[pl.ds(off, ZROWS)])
        return 0
      lax.fori_loop(0, (NCHUNK + NS - 1) // NS, wb, 0)

    tables = ((var_lo, var_hi), (constr_lo, constr_hi))
    outs = ((cagg_lo, cagg_hi), (vagg_lo, vagg_hi))
    for h in range(2):
      zero_acc()
      plsc.subcore_barrier()

      @pl.when(c == 0)
      def _():
        process(tables[0][h], outs[0][h])

      @pl.when(c == 1)
      def _():
        process(tables[1][h], outs[1][h])

  return sc_kernel


_sc_scatter = _sc_scatter_build()


def _mlp_body(vf, cf, va, ca,
              w1a, w1b, b1, g1, bt1, w2, b2,
              w3a, w3b, b3, g2, bt2, w4, b4,
              ov, oc):
  def upd(x_ref, agg_ref, wa, wb, b, g, bt, wo, bo, out_ref):
    h = jnp.dot(x_ref[...], wa[...], preferred_element_type=jnp.float32)
    h = h + jnp.dot(agg_ref[...], wb[...], preferred_element_type=jnp.float32)
    h = h + b[...]
    mu = jnp.mean(h, axis=0, keepdims=True)
    var = jnp.mean(h * h, axis=0, keepdims=True) - mu * mu
    hn = (h - mu) * lax.rsqrt(var + EPS) * g[...] + bt[...]
    hr = jnp.maximum(hn, 0.0)
    out_ref[...] = (x_ref[...] +
                    jnp.dot(hr, wo[...], preferred_element_type=jnp.float32) +
                    bo[...])

  upd(vf, va, w1a, w1b, b1, g1, bt1, w2, b2, ov)
  upd(cf, ca, w3a, w3b, b3, g2, bt2, w4, b4, oc)


def kernel(var_feats, constr_feats, edge_index, edge_attr,
           W1, b1, g1, bt1, W2, b2, W3, b3, g2, bt2, W4, b4):
  src = edge_index[0]
  dst = edge_index[1]
  pad = NE_PAD - N_EDGES
  src_r = jnp.concatenate([src, jnp.zeros((pad,), jnp.int32)]
                          ).reshape(NS, NBLK, EB)
  dst_r = jnp.concatenate([dst, jnp.zeros((pad,), jnp.int32)]
                          ).reshape(NS, NBLK, EB)
  attr_r = jnp.concatenate([edge_attr, jnp.zeros((pad,), jnp.float32)]
                           ).reshape(NS, NBLK * EB)

  vagg_lo, vagg_hi, cagg_lo, cagg_hi = _sc_scatter(
      var_feats[:, :DH], var_feats[:, DH:],
      constr_feats[:, :DH], constr_feats[:, DH:],
      src_r, dst_r, attr_r)
  var_agg = jnp.concatenate([vagg_lo, vagg_hi], axis=1)
  constr_agg = jnp.concatenate([cagg_lo, cagg_hi], axis=1)

  row = lambda v: v.reshape(1, D)
  out_shape = (jax.ShapeDtypeStruct((N_NODES, D), jnp.float32),
               jax.ShapeDtypeStruct((N_NODES, D), jnp.float32))
  var_updated, constr_updated = pl.pallas_call(
      _mlp_body, out_shape=out_shape)(
          var_feats, constr_feats, var_agg, constr_agg,
          W1[:, :D].T, W1[:, D:].T, row(b1), row(g1), row(bt1), W2.T, row(b2),
          W3[:, :D].T, W3[:, D:].T, row(b3), row(g2), row(bt2), W4.T, row(b4))
  return (var_updated, constr_updated)


# speedup vs baseline: 3.2626x; 1.0634x over previous
import functools

import jax
import jax.numpy as jnp
from jax import lax
from jax.experimental import pallas as pl
from jax.experimental.pallas import tpu as pltpu
from jax.experimental.pallas import tpu_sc as plsc

N_NODES = 10000
N_EDGES = 320000
D = 128
DH = D // 2
EPS = 1e-5

NC = 2
NS = 16
EB = 128
NBLK = 160
NE_PAD = NS * NBLK * EB
ZROWS = 40
NCHUNK = N_NODES // ZROWS
NRING = 6
LOOK = 4
CH = 40
NCHB = NBLK // CH


def _sc_scatter_build():
  mesh = plsc.VectorSubcoreMesh(core_axis_name="c", subcore_axis_name="s")
  half = jax.ShapeDtypeStruct((N_NODES, DH), jnp.float32)
  out_t = (half, half, half, half)

  @functools.partial(
      pl.kernel,
      out_type=out_t,
      mesh=mesh,
      compiler_params=pltpu.CompilerParams(use_tc_tiling_on_sc=False),
      scratch_types=[
          pltpu.VMEM((CH, EB), jnp.int32),
          pltpu.VMEM((CH, EB), jnp.int32),
          pltpu.VMEM((CH * EB,), jnp.float32),
          pltpu.VMEM((NRING, EB, DH), jnp.float32),
          pltpu.VMEM((ZROWS, DH), jnp.float32),
          pltpu.VMEM_SHARED((N_NODES, DH), jnp.float32),
          pltpu.SemaphoreType.DMA((NRING,)),
          pltpu.SemaphoreType.DMA((NRING,)),
      ],
  )
  def sc_kernel(var_lo, var_hi, constr_lo, constr_hi,
                src_hbm, dst_hbm, attr_hbm,
                vagg_lo, vagg_hi, cagg_lo, cagg_hi,
                gidx_v, sidx_v, attr_v, rows_v, zbuf_v, acc_sh,
                sem_g, sem_s):
    c = lax.axis_index("c")
    s = lax.axis_index("s")

    def zfill(i, _):
      for g in range(DH // 16):
        zbuf_v[i, pl.ds(g * 16, 16)] = jnp.zeros((16,), jnp.float32)
      return 0
    lax.fori_loop(0, ZROWS, zfill, 0)

    def zero_acc():
      def zchunk(k, _):
        idx = s + k * NS
        @pl.when(idx < NCHUNK)
        def _():
          pltpu.sync_copy(zbuf_v, acc_sh.at[pl.ds(idx * ZROWS, ZROWS)])
        return 0
      lax.fori_loop(0, (NCHUNK + NS - 1) // NS, zchunk, 0)

    def process(table_hbm, g_hbm, s_hbm, out_hbm):
      def g_start(j, b):
        pltpu.async_copy(table_hbm.at[gidx_v.at[j]], rows_v.at[b],
                         sem_g.at[b])

      def g_wait(j, b):
        pltpu.make_async_copy(table_hbm.at[gidx_v.at[j]], rows_v.at[b],
                              sem_g.at[b]).wait()

      def s_start(j, b):
        pltpu.async_copy(rows_v.at[b], acc_sh.at[sidx_v.at[j]],
                         sem_s.at[b], add=True)

      def s_wait(j, b):
        pltpu.make_async_copy(rows_v.at[b], acc_sh.at[sidx_v.at[j]],
                              sem_s.at[b]).wait()

      def chunk(ck, _):
        pltpu.sync_copy(g_hbm.at[s, pl.ds(ck * CH, CH)], gidx_v)
        pltpu.sync_copy(s_hbm.at[s, pl.ds(ck * CH, CH)], sidx_v)
        pltpu.sync_copy(attr_hbm.at[s, pl.ds(ck * CH * EB, CH * EB)], attr_v)

        for jj in range(LOOK):
          g_start(jj, jj)

        def block(j, _):
          b = lax.rem(j, NRING)

          @pl.when(j >= NRING - LOOK)
          def _():
            jd = j - (NRING - LOOK)
            s_wait(jd, lax.rem(jd, NRING))

          @pl.when(j + LOOK < CH)
          def _():
            g_start(j + LOOK, lax.rem(j + LOOK, NRING))

          g_wait(j, b)

          def mul(gr, _):
            base = gr * 16
            a16 = attr_v[pl.ds(j * EB + base, 16)]
            for e16 in range(16):
              a = a16.at[jnp.full((16,), e16, jnp.int32)].get(
                  mode="promise_in_bounds")
              for g in range(DH // 16):
                sl = pl.ds(g * 16, 16)
                rows_v[b, base + e16, sl] = rows_v[b, base + e16, sl] * a
            return 0
          lax.fori_loop(0, EB // 16, mul, 0)

          s_start(j, b)
          return 0
        lax.fori_loop(0, CH, block, 0)
        for jj in range(CH - (NRING - LOOK), CH):
          s_wait(jj, jj % NRING)
        return 0
      lax.fori_loop(0, NCHB, chunk, 0)
      plsc.subcore_barrier()

      def wb(k, _):
        idx = s + k * NS
        @pl.when(idx < NCHUNK)
        def _():
          off = idx * ZROWS
          pltpu.sync_copy(acc_sh.at[pl.ds(off, ZROWS)],
                          out_hbm.at[pl.ds(off, ZROWS)])
        return 0
      lax.fori_loop(0, (NCHUNK + NS - 1) // NS, wb, 0)

    tables = ((var_lo, var_hi), (constr_lo, constr_hi))
    outs = ((cagg_lo, cagg_hi), (vagg_lo, vagg_hi))
    for h in range(2):
      zero_acc()
      plsc.subcore_barrier()

      @pl.when(c == 0)
      def _():
        process(tables[0][h], src_hbm, dst_hbm, outs[0][h])

      @pl.when(c == 1)
      def _():
        process(tables[1][h], dst_hbm, src_hbm, outs[1][h])

  return sc_kernel


_sc_scatter = _sc_scatter_build()


def _mlp_body(vf, cf, va, ca,
              w1a, w1b, b1, g1, bt1, w2, b2,
              w3a, w3b, b3, g2, bt2, w4, b4,
              ov, oc):
  def upd(x_ref, agg_ref, wa, wb, b, g, bt, wo, bo, out_ref):
    h = jnp.dot(x_ref[...], wa[...], preferred_element_type=jnp.float32)
    h = h + jnp.dot(agg_ref[...], wb[...], preferred_element_type=jnp.float32)
    h = h + b[...]
    mu = jnp.mean(h, axis=0, keepdims=True)
    var = jnp.mean(h * h, axis=0, keepdims=True) - mu * mu
    hn = (h - mu) * lax.rsqrt(var + EPS) * g[...] + bt[...]
    hr = jnp.maximum(hn, 0.0)
    out_ref[...] = (x_ref[...] +
                    jnp.dot(hr, wo[...], preferred_element_type=jnp.float32) +
                    bo[...])

  upd(vf, va, w1a, w1b, b1, g1, bt1, w2, b2, ov)
  upd(cf, ca, w3a, w3b, b3, g2, bt2, w4, b4, oc)


def kernel(var_feats, constr_feats, edge_index, edge_attr,
           W1, b1, g1, bt1, W2, b2, W3, b3, g2, bt2, W4, b4):
  src = edge_index[0]
  dst = edge_index[1]
  pad = NE_PAD - N_EDGES
  src_r = jnp.concatenate([src, jnp.zeros((pad,), jnp.int32)]
                          ).reshape(NS, NBLK, EB)
  dst_r = jnp.concatenate([dst, jnp.zeros((pad,), jnp.int32)]
                          ).reshape(NS, NBLK, EB)
  attr_r = jnp.concatenate([edge_attr, jnp.zeros((pad,), jnp.float32)]
                           ).reshape(NS, NBLK * EB)

  vagg_lo, vagg_hi, cagg_lo, cagg_hi = _sc_scatter(
      var_feats[:, :DH], var_feats[:, DH:],
      constr_feats[:, :DH], constr_feats[:, DH:],
      src_r, dst_r, attr_r)
  var_agg = jnp.concatenate([vagg_lo, vagg_hi], axis=1)
  constr_agg = jnp.concatenate([cagg_lo, cagg_hi], axis=1)

  row = lambda v: v.reshape(1, D)
  out_shape = (jax.ShapeDtypeStruct((N_NODES, D), jnp.float32),
               jax.ShapeDtypeStruct((N_NODES, D), jnp.float32))
  var_updated, constr_updated = pl.pallas_call(
      _mlp_body, out_shape=out_shape)(
          var_feats, constr_feats, var_agg, constr_agg,
          W1[:, :D].T, W1[:, D:].T, row(b1), row(g1), row(bt1), W2.T, row(b2),
          W3[:, :D].T, W3[:, D:].T, row(b3), row(g2), row(bt2), W4.T, row(b4))
  return (var_updated, constr_updated)


# speedup vs baseline: 4.8261x; 1.4792x over previous
import functools

import jax
import jax.numpy as jnp
from jax import lax
from jax.experimental import pallas as pl
from jax.experimental.pallas import tpu as pltpu
from jax.experimental.pallas import tpu_sc as plsc

N_NODES = 10000
N_EDGES = 320000
D = 128
DH = D // 2
EPS = 1e-5

NC = 2
NS = 16
EB = 128
NBLK = 160
NE_PAD = NS * NBLK * EB
ZROWS = 40
NCHUNK = N_NODES // ZROWS
NRING = 6
NRING_S = 3
LOOK = 4
CH = 40
NCHB = NBLK // CH

_PH = []
for _g in range(2):
  _PH += [32 * _g + 2 * _i for _i in range(16)]
  _PH += [32 * _g + 2 * _i + 1 for _i in range(16)]
_PERM = _PH + [DH + _x for _x in _PH]


def _sc_scatter_build():
  mesh = plsc.VectorSubcoreMesh(core_axis_name="c", subcore_axis_name="s")
  half = jax.ShapeDtypeStruct((N_NODES, DH), jnp.float32)
  out_t = (half, half, half, half)

  @functools.partial(
      pl.kernel,
      out_type=out_t,
      mesh=mesh,
      compiler_params=pltpu.CompilerParams(use_tc_tiling_on_sc=False, needs_layout_passes=False),
      scratch_types=[
          pltpu.VMEM((CH, EB), jnp.int32),
          pltpu.VMEM((CH, EB), jnp.int32),
          pltpu.VMEM((CH * EB,), jnp.float32),
          pltpu.VMEM((NRING, EB, DH), jnp.bfloat16),
          pltpu.VMEM((NRING_S, EB, DH), jnp.float32),
          pltpu.VMEM((ZROWS, DH), jnp.float32),
          pltpu.VMEM_SHARED((N_NODES, DH), jnp.float32),
          pltpu.SemaphoreType.DMA((NRING,)),
          pltpu.SemaphoreType.DMA((NRING_S,)),
      ],
  )
  def sc_kernel(var_lo, var_hi, constr_lo, constr_hi,
                src_hbm, dst_hbm, attr_hbm,
                vagg_lo, vagg_hi, cagg_lo, cagg_hi,
                gidx_v, sidx_v, attr_v, rows_v, rows_o, zbuf_v, acc_sh,
                sem_g, sem_s):
    c = lax.axis_index("c")
    s = lax.axis_index("s")

    def zfill(i, _):
      for g in range(DH // 16):
        zbuf_v[i, pl.ds(g * 16, 16)] = jnp.zeros((16,), jnp.float32)
      return 0
    lax.fori_loop(0, ZROWS, zfill, 0)

    def zero_acc():
      def zchunk(k, _):
        idx = s + k * NS
        @pl.when(idx < NCHUNK)
        def _():
          pltpu.sync_copy(zbuf_v, acc_sh.at[pl.ds(idx * ZROWS, ZROWS)])
        return 0
      lax.fori_loop(0, (NCHUNK + NS - 1) // NS, zchunk, 0)

    def process(table_hbm, g_hbm, s_hbm, out_hbm):
      def g_start(j, b):
        pltpu.async_copy(table_hbm.at[gidx_v.at[j]], rows_v.at[b],
                         sem_g.at[b])

      def g_wait(j, b):
        pltpu.make_async_copy(table_hbm.at[gidx_v.at[j]], rows_v.at[b],
                              sem_g.at[b]).wait()

      def s_start(j, b):
        pltpu.async_copy(rows_o.at[b], acc_sh.at[sidx_v.at[j]],
                         sem_s.at[b], add=True)

      def s_wait(j, b):
        pltpu.make_async_copy(rows_o.at[b], acc_sh.at[sidx_v.at[j]],
                              sem_s.at[b]).wait()

      def chunk(ck, _):
        pltpu.sync_copy(g_hbm.at[s, pl.ds(ck * CH, CH)], gidx_v)
        pltpu.sync_copy(s_hbm.at[s, pl.ds(ck * CH, CH)], sidx_v)
        pltpu.sync_copy(attr_hbm.at[s, pl.ds(ck * CH * EB, CH * EB)], attr_v)

        for jj in range(LOOK):
          g_start(jj, jj)

        def block(j, _):
          b = lax.rem(j, NRING)
          bs = lax.rem(j, NRING_S)

          @pl.when(j + LOOK < CH)
          def _():
            g_start(j + LOOK, lax.rem(j + LOOK, NRING))

          @pl.when(j >= NRING_S)
          def _():
            s_wait(j - NRING_S, bs)

          g_wait(j, b)

          def mul(gr, _):
            base = gr * 16
            a16 = attr_v[pl.ds(j * EB + base, 16)]
            for e16 in range(16):
              a = a16.at[jnp.full((16,), e16, jnp.int32)].get(
                  mode="promise_in_bounds")
              e = base + e16
              for g in range(DH // 32):
                a32 = rows_v[b, e, pl.ds(g * 32, 32)]
                xe, xo = plsc.unpack(a32, format=plsc.PackFormat.INTERLEAVED)
                rows_o[bs, e, pl.ds(g * 32, 16)] = xe * a
                rows_o[bs, e, pl.ds(g * 32 + 16, 16)] = xo * a
            return 0
          lax.fori_loop(0, EB // 16, mul, 0)

          s_start(j, bs)
          return 0
        lax.fori_loop(0, CH, block, 0)
        for jj in range(CH - NRING_S, CH):
          s_wait(jj, jj % NRING_S)
        return 0
      lax.fori_loop(0, NCHB, chunk, 0)
      plsc.subcore_barrier()

      def wb(k, _):
        idx = s + k * NS
        @pl.when(idx < NCHUNK)
        def _():
          off = idx * ZROWS
          pltpu.sync_copy(acc_sh.at[pl.ds(off, ZROWS)],
                          out_hbm.at[pl.ds(off, ZROWS)])
        return 0
      lax.fori_loop(0, (NCHUNK + NS - 1) // NS, wb, 0)

    tables = ((var_lo, var_hi), (constr_lo, constr_hi))
    outs = ((cagg_lo, cagg_hi), (vagg_lo, vagg_hi))
    for h in range(2):
      zero_acc()
      plsc.subcore_barrier()

      @pl.when(c == 0)
      def _():
        process(tables[0][h], src_hbm, dst_hbm, outs[0][h])

      @pl.when(c == 1)
      def _():
        process(tables[1][h], dst_hbm, src_hbm, outs[1][h])

  return sc_kernel


_sc_scatter = _sc_scatter_build()


def _mlp_body(vf, cf, va, ca,
              w1a, w1b, b1, g1, bt1, w2, b2,
              w3a, w3b, b3, g2, bt2, w4, b4,
              ov, oc):
  def upd(x_ref, agg_ref, wa, wb, b, g, bt, wo, bo, out_ref):
    h = jnp.dot(x_ref[...], wa[...], preferred_element_type=jnp.float32)
    h = h + jnp.dot(agg_ref[...], wb[...], preferred_element_type=jnp.float32)
    h = h + b[...]
    mu = jnp.mean(h, axis=0, keepdims=True)
    var = jnp.mean(h * h, axis=0, keepdims=True) - mu * mu
    hn = (h - mu) * lax.rsqrt(var + EPS) * g[...] + bt[...]
    hr = jnp.maximum(hn, 0.0)
    out_ref[...] = (x_ref[...] +
                    jnp.dot(hr, wo[...], preferred_element_type=jnp.float32) +
                    bo[...])

  upd(vf, va, w1a, w1b, b1, g1, bt1, w2, b2, ov)
  upd(cf, ca, w3a, w3b, b3, g2, bt2, w4, b4, oc)


def kernel(var_feats, constr_feats, edge_index, edge_attr,
           W1, b1, g1, bt1, W2, b2, W3, b3, g2, bt2, W4, b4):
  src = edge_index[0]
  dst = edge_index[1]
  pad = NE_PAD - N_EDGES
  src_r = jnp.concatenate([src, jnp.zeros((pad,), jnp.int32)]
                          ).reshape(NS, NBLK, EB)
  dst_r = jnp.concatenate([dst, jnp.zeros((pad,), jnp.int32)]
                          ).reshape(NS, NBLK, EB)
  attr_r = jnp.concatenate([edge_attr, jnp.zeros((pad,), jnp.float32)]
                           ).reshape(NS, NBLK * EB)

  bf = jnp.bfloat16
  vagg_lo, vagg_hi, cagg_lo, cagg_hi = _sc_scatter(
      var_feats[:, :DH].astype(bf), var_feats[:, DH:].astype(bf),
      constr_feats[:, :DH].astype(bf), constr_feats[:, DH:].astype(bf),
      src_r, dst_r, attr_r)
  var_agg = jnp.concatenate([vagg_lo, vagg_hi], axis=1)
  constr_agg = jnp.concatenate([cagg_lo, cagg_hi], axis=1)

  row = lambda v: v.reshape(1, D)
  out_shape = (jax.ShapeDtypeStruct((N_NODES, D), jnp.float32),
               jax.ShapeDtypeStruct((N_NODES, D), jnp.float32))
  perm = jnp.array(_PERM, dtype=jnp.int32)
  var_updated, constr_updated = pl.pallas_call(
      _mlp_body, out_shape=out_shape)(
          var_feats, constr_feats, var_agg, constr_agg,
          W1[:, :D].T, W1[:, D:].T[perm],
          row(b1), row(g1), row(bt1), W2.T, row(b2),
          W3[:, :D].T, W3[:, D:].T[perm],
          row(b3), row(g2), row(bt2), W4.T, row(b4))
  return (var_updated, constr_updated)
